# Initial kernel scaffold; baseline (speedup 1.0000x reference)
import jax
import jax.numpy as jnp
from jax.experimental import pallas as pl


def kernel(x, edge_index, action_nodeIDs, W1, al1, ar1, b1, W2, al2, ar2, b2, W3, al3, ar3, b3, W4, al4, ar4, b4, Wf, bf):
    raise NotImplementedError("write your pallas kernel here")


# speedup vs baseline: 22.5124x; 22.5124x over previous
import functools
import jax
import jax.numpy as jnp
from jax import lax
from jax.experimental import pallas as pl
from jax.experimental.pallas import tpu as pltpu
from jax.experimental.pallas import tpu_sc as plsc

B, N, E = 4, 10000, 160000
D_IN, HID, HEADS = 128, 128, 4
NP = 10240
ET = E + N
CH = 128
EPT = 10752
EP = EPT * 16
NCHUNK = EPT // CH
ROWS_T = NP // 16
HI = jax.lax.Precision.DEFAULT
HIX = jax.lax.Precision.HIGHEST


def _leaky(x):
    return jnp.where(x >= 0, x, 0.2 * x)


def _allmax(v):
    for shft in (1, 2, 4, 8):
        idx = jnp.arange(16, dtype=jnp.int32) ^ shft
        perm = lax.gather(
            v, idx[:, None],
            lax.GatherDimensionNumbers(offset_dims=(),
                                       collapsed_slice_dims=(0,),
                                       start_index_map=(0,)),
            slice_sizes=(1,),
            mode=lax.GatherScatterMode.PROMISE_IN_BOUNDS)
        v = jnp.maximum(v, perm)
    return v


def _make_sc(H):
    mesh = plsc.VectorSubcoreMesh(core_axis_name="c", subcore_axis_name="s",
                                  num_cores=2, num_subcores=16)

    def body(el_hbm, er_hbm, src_hbm, dst_hbm, z_hbm, out_hbm, s_hbm,
             spart_hbm,
             el_v, er_v, spart, sred, tmp_v, srcv, dstv, gidxv,
             rows, out_acc):
        c = lax.axis_index("c")
        t = lax.axis_index("s")
        lanes = lax.iota(jnp.int32, 16)
        lane_masks = [lanes == kk for kk in range(16)]

        def gh_iter(gh, _):
            g = c * 2 + gh // H
            h = gh % H
            ghg = g * H + h
            zb = g * NP * H + h

            pltpu.sync_copy(el_hbm.at[ghg], el_v)
            pltpu.sync_copy(er_hbm.at[ghg], er_v)

            def mx_step(i, acc):
                return jnp.maximum(acc, el_v[pl.ds(i * 16, 16)])
            mxv = lax.fori_loop(0, 625, mx_step,
                                jnp.full((16,), -jnp.inf, jnp.float32))
            mx = _allmax(mxv)

            def z1(i, _):
                spart[pl.ds(i * 16, 16)] = jnp.zeros((16,), jnp.float32)
                return ()
            lax.fori_loop(0, NP // 16, z1, ())

            for r in range(16):
                for j in range(8):
                    rows[r, pl.ds(j * 16, 16)] = jnp.zeros((16,), jnp.float32)

            def z2(i, _):
                pltpu.sync_copy(rows.at[pl.ds(0, 16), :],
                                out_acc.at[pl.ds(t * ROWS_T + i * 16, 16), :])
                return ()
            lax.fori_loop(0, ROWS_T // 16, z2, ())
            plsc.subcore_barrier()

            def chunk(i, _):
                off = t * EPT + i * CH
                pltpu.sync_copy(src_hbm.at[g, pl.ds(off, CH)], srcv)
                pltpu.sync_copy(dst_hbm.at[g, pl.ds(off, CH)], dstv)
                for j in range(CH // 16):
                    sv = srcv[pl.ds(j * 16, 16)]
                    gidxv[pl.ds(j * 16, 16)] = sv * H + zb
                pltpu.sync_copy(z_hbm.at[gidxv], rows)
                for j in range(CH // 16):
                    sv = srcv[pl.ds(j * 16, 16)]
                    dv = dstv[pl.ds(j * 16, 16)]
                    el_s = plsc.load_gather(el_v, [sv])
                    er_d = plsc.load_gather(er_v, [dv])
                    e = _leaky(el_s + er_d)
                    sh = _leaky(er_d + mx)
                    gid = off + j * 16 + lax.iota(jnp.int32, 16)
                    ex = jnp.where(gid < ET, jnp.exp(e - sh), 0.0)
                    for mk in lane_masks:
                        plsc.addupdate_scatter(spart, [dv], ex, mask=mk)
                    for kk in range(16):
                        k = j * 16 + kk
                        a = jnp.broadcast_to(ex[kk], (16,))
                        for jj in range(8):
                            rows[k, pl.ds(jj * 16, 16)] = (
                                rows[k, pl.ds(jj * 16, 16)] * a)
                pltpu.sync_copy(rows, out_acc.at[dstv], add=True)
                return ()
            lax.fori_loop(0, NCHUNK, chunk, ())

            pltpu.sync_copy(spart, spart_hbm.at[c, t])
            plsc.subcore_barrier()

            def z3(i, _):
                sred[pl.ds(i * 16, 16)] = jnp.zeros((16,), jnp.float32)
                return ()
            lax.fori_loop(0, ROWS_T // 16, z3, ())
            for tt in range(16):
                pltpu.sync_copy(spart_hbm.at[c, tt, pl.ds(t * ROWS_T, ROWS_T)],
                                tmp_v)

                def sr(i, _):
                    sred[pl.ds(i * 16, 16)] = (sred[pl.ds(i * 16, 16)]
                                               + tmp_v[pl.ds(i * 16, 16)])
                    return ()
                lax.fori_loop(0, ROWS_T // 16, sr, ())
            pltpu.sync_copy(sred, s_hbm.at[ghg, pl.ds(t * ROWS_T, ROWS_T)])
            pltpu.sync_copy(out_acc.at[pl.ds(t * ROWS_T, ROWS_T), :],
                            out_hbm.at[pl.ds(ghg * NP + t * ROWS_T,
                                             ROWS_T), :])
            plsc.subcore_barrier()
            return ()

        lax.fori_loop(0, 2 * H, gh_iter, ())

    return pl.kernel(
        body,
        out_type=(
            jax.ShapeDtypeStruct((B * H * NP, HID), jnp.float32),
            jax.ShapeDtypeStruct((B * H, NP), jnp.float32),
            jax.ShapeDtypeStruct((2, 16, NP), jnp.float32),
        ),
        mesh=mesh,
        compiler_params=pltpu.CompilerParams(needs_layout_passes=False),
        scratch_types=[
            pltpu.VMEM((NP,), jnp.float32),
            pltpu.VMEM((NP,), jnp.float32),
            pltpu.VMEM((NP,), jnp.float32),
            pltpu.VMEM((ROWS_T,), jnp.float32),
            pltpu.VMEM((ROWS_T,), jnp.float32),
            pltpu.VMEM((CH,), jnp.int32),
            pltpu.VMEM((CH,), jnp.int32),
            pltpu.VMEM((CH,), jnp.int32),
            pltpu.VMEM((CH, HID), jnp.float32),
            pltpu.VMEM_SHARED((NP, HID), jnp.float32),
        ],
    )


_sc_cache = {}


def _sc_edge(Hh):
    if Hh not in _sc_cache:
        _sc_cache[Hh] = _make_sc(Hh)
    return _sc_cache[Hh]


NB = 1280


def _tc1_body(x_ref, w_ref, al_ref, ar_ref, z_ref, el_ref, er_ref):
    xb = x_ref[0]
    z = jnp.dot(xb, w_ref[...], preferred_element_type=jnp.float32,
                precision=HI)
    zh = z.reshape(NB, HEADS, HID)
    z_ref[0] = zh
    el = jnp.sum(zh * al_ref[...][None], axis=-1)
    er = jnp.sum(zh * ar_ref[...][None], axis=-1)
    el_ref[0] = el.T
    er_ref[0] = er.T


def _tc1(x, W1, al1, ar1):
    return pl.pallas_call(
        _tc1_body,
        grid=(B, NP // NB),
        in_specs=[
            pl.BlockSpec((1, NB, D_IN), lambda b, i: (b, i, 0)),
            pl.BlockSpec((D_IN, HEADS * HID), lambda b, i: (0, 0)),
            pl.BlockSpec((HEADS, HID), lambda b, i: (0, 0)),
            pl.BlockSpec((HEADS, HID), lambda b, i: (0, 0)),
        ],
        out_specs=[
            pl.BlockSpec((1, NB, HEADS, HID), lambda b, i: (b, i, 0, 0)),
            pl.BlockSpec((1, HEADS, NB), lambda b, i: (b, 0, i)),
            pl.BlockSpec((1, HEADS, NB), lambda b, i: (b, 0, i)),
        ],
        out_shape=[
            jax.ShapeDtypeStruct((B, NP, HEADS, HID), jnp.float32),
            jax.ShapeDtypeStruct((B, HEADS, NP), jnp.float32),
            jax.ShapeDtypeStruct((B, HEADS, NP), jnp.float32),
        ],
    )(x, W1, al1, ar1)


def _make_tc2(Hp):
    def body(un_ref, s_ref, b_ref, w_ref, al_ref, ar_ref,
             z_ref, el_ref, er_ref):
        acc = jnp.zeros((NB, HID), jnp.float32)
        for h in range(Hp):
            u = un_ref[0, h]
            sv = s_ref[0, h]
            act = jnp.maximum(u * (1.0 / (sv + 1e-9))[:, None]
                              + b_ref[h][None], 0.0)
            acc = acc + jnp.dot(act, w_ref[h],
                                preferred_element_type=jnp.float32,
                                precision=HI)
        zh = acc.reshape(NB, 1, HID)
        z_ref[0] = zh
        el_ref[0] = jnp.sum(acc * al_ref[...], axis=-1)[None]
        er_ref[0] = jnp.sum(acc * ar_ref[...], axis=-1)[None]

    def run(un, s, bprev, W, al, ar):
        un4 = un.reshape(B, Hp, NP, HID)
        s3 = s.reshape(B, Hp, NP)
        return pl.pallas_call(
            body,
            grid=(B, NP // NB),
            in_specs=[
                pl.BlockSpec((1, Hp, NB, HID), lambda b, i: (b, 0, i, 0)),
                pl.BlockSpec((1, Hp, NB), lambda b, i: (b, 0, i)),
                pl.BlockSpec((Hp, HID), lambda b, i: (0, 0)),
                pl.BlockSpec((Hp, HID, HID), lambda b, i: (0, 0, 0)),
                pl.BlockSpec((1, HID), lambda b, i: (0, 0)),
                pl.BlockSpec((1, HID), lambda b, i: (0, 0)),
            ],
            out_specs=[
                pl.BlockSpec((1, NB, 1, HID), lambda b, i: (b, i, 0, 0)),
                pl.BlockSpec((1, 1, NB), lambda b, i: (b, 0, i)),
                pl.BlockSpec((1, 1, NB), lambda b, i: (b, 0, i)),
            ],
            out_shape=[
                jax.ShapeDtypeStruct((B, NP, 1, HID), jnp.float32),
                jax.ShapeDtypeStruct((B, 1, NP), jnp.float32),
                jax.ShapeDtypeStruct((B, 1, NP), jnp.float32),
            ],
        )(un4, s3, bprev.reshape(Hp, HID), W.reshape(Hp, HID, HID), al, ar)
    return run


_tc2_h4 = _make_tc2(HEADS)
_tc2_h1 = _make_tc2(1)


def _tc3_body(un_ref, s_ref, b_ref, aid_ref, hs_ref):
    bi = pl.program_id(0)
    i = pl.program_id(1)
    u = un_ref[0, 0]
    sv = s_ref[0, 0]
    h3 = jnp.maximum(u * (1.0 / (sv + 1e-9))[:, None] + b_ref[...], 0.0)
    rel = aid_ref[bi] - i * NB
    onehot = (lax.broadcasted_iota(jnp.int32, (1, NB), 1) == rel
              ).astype(jnp.float32)
    contrib = jnp.dot(onehot, h3, preferred_element_type=jnp.float32,
                      precision=HIX)

    @pl.when(i == 0)
    def _():
        hs_ref[...] = jnp.zeros_like(hs_ref)
    hs_ref[0] += contrib


def _tc3(un, s, b3, aids):
    un4 = un.reshape(B, 1, NP, HID)
    s3 = s.reshape(B, 1, NP)
    return pl.pallas_call(
        _tc3_body,
        grid=(B, NP // NB),
        in_specs=[
            pl.BlockSpec((1, 1, NB, HID), lambda b, i: (b, 0, i, 0)),
            pl.BlockSpec((1, 1, NB), lambda b, i: (b, 0, i)),
            pl.BlockSpec((1, HID), lambda b, i: (0, 0)),
            pl.BlockSpec(memory_space=pltpu.SMEM),
        ],
        out_specs=pl.BlockSpec((1, 1, HID), lambda b, i: (b, 0, 0)),
        out_shape=jax.ShapeDtypeStruct((B, 1, HID), jnp.float32),
    )(un4, s3, b3.reshape(1, HID), aids).reshape(B, HID)


def _tc4_body(hs_ref, w4_ref, b4_ref, wf_ref, bf_ref, out_ref):
    h4 = jnp.maximum(jnp.dot(hs_ref[...], w4_ref[...],
                             preferred_element_type=jnp.float32,
                             precision=HI) + b4_ref[...][None], 0.0)
    out_ref[...] = jnp.dot(h4, wf_ref[...],
                           preferred_element_type=jnp.float32,
                           precision=HI) + bf_ref[...][None]


def _tc4(hs, W4, b4, Wf, bf):
    return pl.pallas_call(
        _tc4_body,
        out_shape=jax.ShapeDtypeStruct((B, 1), jnp.float32),
    )(hs, W4, b4, Wf, bf)


@jax.jit
def kernel(x, edge_index, action_nodeIDs, W1, al1, ar1, b1, W2, al2, ar2, b2,
           W3, al3, ar3, b3, W4, al4, ar4, b4, Wf, bf):
    loop = jnp.broadcast_to(jnp.arange(N, dtype=jnp.int32)[None], (B, N))
    src = jnp.concatenate([edge_index[:, 0, :], loop], axis=1)
    dst = jnp.concatenate([edge_index[:, 1, :], loop], axis=1)
    src = jnp.pad(src, ((0, 0), (0, EP - ET)))
    dst = jnp.pad(dst, ((0, 0), (0, EP - ET)))

    z1, el1, er1 = _tc1(x, W1, al1.reshape(HEADS, HID), ar1.reshape(HEADS, HID))
    un1, s1, _ = _sc_edge(HEADS)(el1.reshape(B * HEADS, NP),
                              er1.reshape(B * HEADS, NP),
                              src, dst, z1.reshape(B * NP * HEADS, HID))
    z2, el2, er2 = _tc2_h4(un1, s1, b1, W2, al2, ar2)
    un2, s2, _ = _sc_edge(1)(el2.reshape(B, NP), er2.reshape(B, NP),
                          src, dst, z2.reshape(B * NP, HID))
    z3, el3, er3 = _tc2_h1(un2, s2, b2, W3, al3, ar3)
    un3, s3, _ = _sc_edge(1)(el3.reshape(B, NP), er3.reshape(B, NP),
                          src, dst, z3.reshape(B * NP, HID))
    hs = _tc3(un3, s3, b3, action_nodeIDs)
    return _tc4(hs, W4, b4, Wf, bf)


# speedup vs baseline: 26.6768x; 1.1850x over previous
import functools
import jax
import jax.numpy as jnp
from jax import lax
from jax.experimental import pallas as pl
from jax.experimental.pallas import tpu as pltpu
from jax.experimental.pallas import tpu_sc as plsc

B, N, E = 4, 10000, 160000
D_IN, HID, HEADS = 128, 128, 4
NP = 10240
ET = E + N
CH = 64
EPT = 10752
EP = EPT * 16
NCHUNK = EPT // CH
ROWS_T = NP // 16
HI = jax.lax.Precision.DEFAULT
HIX = jax.lax.Precision.HIGHEST


def _leaky(x):
    return jnp.where(x >= 0, x, 0.2 * x)


def _allmax(v):
    for shft in (1, 2, 4, 8):
        idx = jnp.arange(16, dtype=jnp.int32) ^ shft
        perm = lax.gather(
            v, idx[:, None],
            lax.GatherDimensionNumbers(offset_dims=(),
                                       collapsed_slice_dims=(0,),
                                       start_index_map=(0,)),
            slice_sizes=(1,),
            mode=lax.GatherScatterMode.PROMISE_IN_BOUNDS)
        v = jnp.maximum(v, perm)
    return v


def _make_sc(H):
    mesh = plsc.VectorSubcoreMesh(core_axis_name="c", subcore_axis_name="s",
                                  num_cores=2, num_subcores=16)

    def body(el_hbm, er_hbm, src_hbm, dst_hbm, z_hbm, out_hbm, s_hbm,
             spart_hbm,
             el_v, er_v, spart, sred, tmp_v,
             srcv0, dstv0, gidxv0, rows0, sem0,
             srcv1, dstv1, gidxv1, rows1, sem1,
             out_acc):
        c = lax.axis_index("c")
        t = lax.axis_index("s")
        lanes = lax.iota(jnp.int32, 16)
        lane_masks = [lanes == kk for kk in range(16)]

        def gh_iter(gh, _):
            g = c * 2 + gh // H
            h = gh % H
            ghg = g * H + h
            zb = g * NP * H + h

            pltpu.sync_copy(el_hbm.at[ghg], el_v)
            pltpu.sync_copy(er_hbm.at[ghg], er_v)

            def mx_step(i, acc):
                return jnp.maximum(acc, el_v[pl.ds(i * 16, 16)])
            mxv = lax.fori_loop(0, 625, mx_step,
                                jnp.full((16,), -jnp.inf, jnp.float32))
            mx = _allmax(mxv)

            def z1(i, _):
                spart[pl.ds(i * 16, 16)] = jnp.zeros((16,), jnp.float32)
                return ()
            lax.fori_loop(0, NP // 16, z1, ())

            for r in range(16):
                for j in range(8):
                    rows0[r, pl.ds(j * 16, 16)] = jnp.zeros((16,), jnp.float32)

            def z2(i, _):
                pltpu.sync_copy(rows0.at[pl.ds(0, 16), :],
                                out_acc.at[pl.ds(t * ROWS_T + i * 16, 16), :])
                return ()
            lax.fori_loop(0, ROWS_T // 16, z2, ())
            plsc.subcore_barrier()

            def issue(i, sb, db, gb, rb, sem):
                off = t * EPT + i * CH
                pltpu.sync_copy(src_hbm.at[g, pl.ds(off, CH)], sb)
                pltpu.sync_copy(dst_hbm.at[g, pl.ds(off, CH)], db)
                for j in range(CH // 16):
                    gb[pl.ds(j * 16, 16)] = sb[pl.ds(j * 16, 16)] * H + zb
                pltpu.async_copy(z_hbm.at[gb], rb, sem)

            def process(i, sb, db, gb, rb, sem):
                pltpu.make_async_copy(z_hbm.at[gb], rb, sem).wait()
                off = t * EPT + i * CH
                for j in range(CH // 16):
                    sv = sb[pl.ds(j * 16, 16)]
                    dv = db[pl.ds(j * 16, 16)]
                    el_s = plsc.load_gather(el_v, [sv])
                    er_d = plsc.load_gather(er_v, [dv])
                    e = _leaky(el_s + er_d)
                    sh = _leaky(er_d + mx)
                    gid = off + j * 16 + lax.iota(jnp.int32, 16)
                    ex = jnp.where(gid < ET, jnp.exp(e - sh), 0.0)
                    for mk in lane_masks:
                        plsc.addupdate_scatter(spart, [dv], ex, mask=mk)
                    for kk in range(16):
                        k = j * 16 + kk
                        a = jnp.broadcast_to(ex[kk], (16,))
                        for jj in range(8):
                            rb[k, pl.ds(jj * 16, 16)] = (
                                rb[k, pl.ds(jj * 16, 16)] * a)
                pltpu.sync_copy(rb, out_acc.at[db], add=True)

            issue(0, srcv0, dstv0, gidxv0, rows0, sem0)

            def piter(ii, _):
                i0 = ii * 2
                issue(i0 + 1, srcv1, dstv1, gidxv1, rows1, sem1)
                process(i0, srcv0, dstv0, gidxv0, rows0, sem0)

                @pl.when(ii < NCHUNK // 2 - 1)
                def _():
                    issue(i0 + 2, srcv0, dstv0, gidxv0, rows0, sem0)
                process(i0 + 1, srcv1, dstv1, gidxv1, rows1, sem1)
                return ()
            lax.fori_loop(0, NCHUNK // 2, piter, ())

            pltpu.sync_copy(spart, spart_hbm.at[c, t])
            plsc.subcore_barrier()

            def z3(i, _):
                sred[pl.ds(i * 16, 16)] = jnp.zeros((16,), jnp.float32)
                return ()
            lax.fori_loop(0, ROWS_T // 16, z3, ())
            for tt in range(16):
                pltpu.sync_copy(spart_hbm.at[c, tt, pl.ds(t * ROWS_T, ROWS_T)],
                                tmp_v)

                def sr(i, _):
                    sred[pl.ds(i * 16, 16)] = (sred[pl.ds(i * 16, 16)]
                                               + tmp_v[pl.ds(i * 16, 16)])
                    return ()
                lax.fori_loop(0, ROWS_T // 16, sr, ())
            pltpu.sync_copy(sred, s_hbm.at[ghg, pl.ds(t * ROWS_T, ROWS_T)])
            pltpu.sync_copy(out_acc.at[pl.ds(t * ROWS_T, ROWS_T), :],
                            out_hbm.at[pl.ds(ghg * NP + t * ROWS_T,
                                             ROWS_T), :])
            plsc.subcore_barrier()
            return ()

        lax.fori_loop(0, 2 * H, gh_iter, ())

    return pl.kernel(
        body,
        out_type=(
            jax.ShapeDtypeStruct((B * H * NP, HID), jnp.float32),
            jax.ShapeDtypeStruct((B * H, NP), jnp.float32),
            jax.ShapeDtypeStruct((2, 16, NP), jnp.float32),
        ),
        mesh=mesh,
        compiler_params=pltpu.CompilerParams(needs_layout_passes=False),
        scratch_types=[
            pltpu.VMEM((NP,), jnp.float32),
            pltpu.VMEM((NP,), jnp.float32),
            pltpu.VMEM((NP,), jnp.float32),
            pltpu.VMEM((ROWS_T,), jnp.float32),
            pltpu.VMEM((ROWS_T,), jnp.float32),
            pltpu.VMEM((CH,), jnp.int32),
            pltpu.VMEM((CH,), jnp.int32),
            pltpu.VMEM((CH,), jnp.int32),
            pltpu.VMEM((CH, HID), jnp.float32),
            pltpu.SemaphoreType.DMA,
            pltpu.VMEM((CH,), jnp.int32),
            pltpu.VMEM((CH,), jnp.int32),
            pltpu.VMEM((CH,), jnp.int32),
            pltpu.VMEM((CH, HID), jnp.float32),
            pltpu.SemaphoreType.DMA,
            pltpu.VMEM_SHARED((NP, HID), jnp.float32),
        ],
    )


_sc_cache = {}


def _sc_edge(Hh):
    if Hh not in _sc_cache:
        _sc_cache[Hh] = _make_sc(Hh)
    return _sc_cache[Hh]


NB = 1280


def _tc1_body(x_ref, w_ref, al_ref, ar_ref, z_ref, el_ref, er_ref):
    xb = x_ref[0]
    z = jnp.dot(xb, w_ref[...], preferred_element_type=jnp.float32,
                precision=HI)
    zh = z.reshape(NB, HEADS, HID)
    z_ref[0] = zh
    el = jnp.sum(zh * al_ref[...][None], axis=-1)
    er = jnp.sum(zh * ar_ref[...][None], axis=-1)
    el_ref[0] = el.T
    er_ref[0] = er.T


def _tc1(x, W1, al1, ar1):
    return pl.pallas_call(
        _tc1_body,
        grid=(B, NP // NB),
        in_specs=[
            pl.BlockSpec((1, NB, D_IN), lambda b, i: (b, i, 0)),
            pl.BlockSpec((D_IN, HEADS * HID), lambda b, i: (0, 0)),
            pl.BlockSpec((HEADS, HID), lambda b, i: (0, 0)),
            pl.BlockSpec((HEADS, HID), lambda b, i: (0, 0)),
        ],
        out_specs=[
            pl.BlockSpec((1, NB, HEADS, HID), lambda b, i: (b, i, 0, 0)),
            pl.BlockSpec((1, HEADS, NB), lambda b, i: (b, 0, i)),
            pl.BlockSpec((1, HEADS, NB), lambda b, i: (b, 0, i)),
        ],
        out_shape=[
            jax.ShapeDtypeStruct((B, NP, HEADS, HID), jnp.float32),
            jax.ShapeDtypeStruct((B, HEADS, NP), jnp.float32),
            jax.ShapeDtypeStruct((B, HEADS, NP), jnp.float32),
        ],
    )(x, W1, al1, ar1)


def _make_tc2(Hp):
    def body(un_ref, s_ref, b_ref, w_ref, al_ref, ar_ref,
             z_ref, el_ref, er_ref):
        acc = jnp.zeros((NB, HID), jnp.float32)
        for h in range(Hp):
            u = un_ref[0, h]
            sv = s_ref[0, h]
            act = jnp.maximum(u * (1.0 / (sv + 1e-9))[:, None]
                              + b_ref[h][None], 0.0)
            acc = acc + jnp.dot(act, w_ref[h],
                                preferred_element_type=jnp.float32,
                                precision=HI)
        zh = acc.reshape(NB, 1, HID)
        z_ref[0] = zh
        el_ref[0] = jnp.sum(acc * al_ref[...], axis=-1)[None]
        er_ref[0] = jnp.sum(acc * ar_ref[...], axis=-1)[None]

    def run(un, s, bprev, W, al, ar):
        un4 = un.reshape(B, Hp, NP, HID)
        s3 = s.reshape(B, Hp, NP)
        return pl.pallas_call(
            body,
            grid=(B, NP // NB),
            in_specs=[
                pl.BlockSpec((1, Hp, NB, HID), lambda b, i: (b, 0, i, 0)),
                pl.BlockSpec((1, Hp, NB), lambda b, i: (b, 0, i)),
                pl.BlockSpec((Hp, HID), lambda b, i: (0, 0)),
                pl.BlockSpec((Hp, HID, HID), lambda b, i: (0, 0, 0)),
                pl.BlockSpec((1, HID), lambda b, i: (0, 0)),
                pl.BlockSpec((1, HID), lambda b, i: (0, 0)),
            ],
            out_specs=[
                pl.BlockSpec((1, NB, 1, HID), lambda b, i: (b, i, 0, 0)),
                pl.BlockSpec((1, 1, NB), lambda b, i: (b, 0, i)),
                pl.BlockSpec((1, 1, NB), lambda b, i: (b, 0, i)),
            ],
            out_shape=[
                jax.ShapeDtypeStruct((B, NP, 1, HID), jnp.float32),
                jax.ShapeDtypeStruct((B, 1, NP), jnp.float32),
                jax.ShapeDtypeStruct((B, 1, NP), jnp.float32),
            ],
        )(un4, s3, bprev.reshape(Hp, HID), W.reshape(Hp, HID, HID), al, ar)
    return run


_tc2_h4 = _make_tc2(HEADS)
_tc2_h1 = _make_tc2(1)


def _tc3_body(un_ref, s_ref, b_ref, aid_ref, hs_ref):
    bi = pl.program_id(0)
    i = pl.program_id(1)
    u = un_ref[0, 0]
    sv = s_ref[0, 0]
    h3 = jnp.maximum(u * (1.0 / (sv + 1e-9))[:, None] + b_ref[...], 0.0)
    rel = aid_ref[bi] - i * NB
    onehot = (lax.broadcasted_iota(jnp.int32, (1, NB), 1) == rel
              ).astype(jnp.float32)
    contrib = jnp.dot(onehot, h3, preferred_element_type=jnp.float32,
                      precision=HIX)

    @pl.when(i == 0)
    def _():
        hs_ref[...] = jnp.zeros_like(hs_ref)
    hs_ref[0] += contrib


def _tc3(un, s, b3, aids):
    un4 = un.reshape(B, 1, NP, HID)
    s3 = s.reshape(B, 1, NP)
    return pl.pallas_call(
        _tc3_body,
        grid=(B, NP // NB),
        in_specs=[
            pl.BlockSpec((1, 1, NB, HID), lambda b, i: (b, 0, i, 0)),
            pl.BlockSpec((1, 1, NB), lambda b, i: (b, 0, i)),
            pl.BlockSpec((1, HID), lambda b, i: (0, 0)),
            pl.BlockSpec(memory_space=pltpu.SMEM),
        ],
        out_specs=pl.BlockSpec((1, 1, HID), lambda b, i: (b, 0, 0)),
        out_shape=jax.ShapeDtypeStruct((B, 1, HID), jnp.float32),
    )(un4, s3, b3.reshape(1, HID), aids).reshape(B, HID)


def _tc4_body(hs_ref, w4_ref, b4_ref, wf_ref, bf_ref, out_ref):
    h4 = jnp.maximum(jnp.dot(hs_ref[...], w4_ref[...],
                             preferred_element_type=jnp.float32,
                             precision=HI) + b4_ref[...][None], 0.0)
    out_ref[...] = jnp.dot(h4, wf_ref[...],
                           preferred_element_type=jnp.float32,
                           precision=HI) + bf_ref[...][None]


def _tc4(hs, W4, b4, Wf, bf):
    return pl.pallas_call(
        _tc4_body,
        out_shape=jax.ShapeDtypeStruct((B, 1), jnp.float32),
    )(hs, W4, b4, Wf, bf)


@jax.jit
def kernel(x, edge_index, action_nodeIDs, W1, al1, ar1, b1, W2, al2, ar2, b2,
           W3, al3, ar3, b3, W4, al4, ar4, b4, Wf, bf):
    loop = jnp.broadcast_to(jnp.arange(N, dtype=jnp.int32)[None], (B, N))
    src = jnp.concatenate([edge_index[:, 0, :], loop], axis=1)
    dst = jnp.concatenate([edge_index[:, 1, :], loop], axis=1)
    src = jnp.pad(src, ((0, 0), (0, EP - ET)))
    dst = jnp.pad(dst, ((0, 0), (0, EP - ET)))

    z1, el1, er1 = _tc1(x, W1, al1.reshape(HEADS, HID), ar1.reshape(HEADS, HID))
    un1, s1, _ = _sc_edge(HEADS)(el1.reshape(B * HEADS, NP),
                              er1.reshape(B * HEADS, NP),
                              src, dst, z1.reshape(B * NP * HEADS, HID))
    z2, el2, er2 = _tc2_h4(un1, s1, b1, W2, al2, ar2)
    un2, s2, _ = _sc_edge(1)(el2.reshape(B, NP), er2.reshape(B, NP),
                          src, dst, z2.reshape(B * NP, HID))
    z3, el3, er3 = _tc2_h1(un2, s2, b2, W3, al3, ar3)
    un3, s3, _ = _sc_edge(1)(el3.reshape(B, NP), er3.reshape(B, NP),
                          src, dst, z3.reshape(B * NP, HID))
    hs = _tc3(un3, s3, b3, action_nodeIDs)
    return _tc4(hs, W4, b4, Wf, bf)
